# baseline (speedup 1.0000x reference)
import functools

import jax
import jax.numpy as jnp
from jax import lax
from jax.experimental import pallas as pl
from jax.experimental.pallas import tpu as pltpu
from jax.experimental.pallas import tpu_sc as plsc

NC = 2
NS = 16
NW = NC * NS

B = 4096 * 200
D = 64
B_PER_W = B // NW
CHUNK = 128
NSTEP = B_PER_W // CHUNK

_mesh = plsc.VectorSubcoreMesh(core_axis_name="c", subcore_axis_name="s")


@functools.partial(
    pl.kernel,
    out_type=jax.ShapeDtypeStruct((B, D), jnp.float32),
    mesh=_mesh,
    scratch_types=[
        pltpu.VMEM((NSTEP, CHUNK), jnp.int32),
        pltpu.VMEM((CHUNK, D), jnp.float32),
        pltpu.SemaphoreType.DMA,
    ],
    compiler_params=pltpu.CompilerParams(use_tc_tiling_on_sc=False),
)
def _embed_gather(idx_hbm, table_hbm, out_hbm, idx_v, rows_v, sem):
    wid = lax.axis_index("s") * NC + lax.axis_index("c")
    base = wid * B_PER_W
    pltpu.sync_copy(idx_hbm.at[pl.ds(wid * NSTEP, NSTEP)], idx_v)

    @pl.loop(0, NSTEP)
    def _(g):
        pltpu.async_copy(table_hbm.at[idx_v.at[g]], rows_v, sem).wait()
        pltpu.sync_copy(rows_v, out_hbm.at[pl.ds(base + g * CHUNK, CHUNK)])


def kernel(token_ids, weight):
    idx = token_ids.reshape(NW * NSTEP, CHUNK).astype(jnp.int32)
    out = _embed_gather(idx, weight)
    return out.reshape(*token_ids.shape, D)


# speedup vs baseline: 1.1147x; 1.1147x over previous
import functools

import jax
import jax.numpy as jnp
from jax import lax
from jax.experimental import pallas as pl
from jax.experimental.pallas import tpu as pltpu
from jax.experimental.pallas import tpu_sc as plsc

NC = 2
NS = 16
NW = NC * NS

B = 4096 * 200
D = 64
B_PER_W = B // NW
CHUNK = 128
NSTEP = B_PER_W // CHUNK
NBUF = 8

_mesh = plsc.VectorSubcoreMesh(core_axis_name="c", subcore_axis_name="s")


@functools.partial(
    pl.kernel,
    out_type=jax.ShapeDtypeStruct((B, D), jnp.float32),
    mesh=_mesh,
    scratch_types=[
        pltpu.VMEM((NSTEP, CHUNK), jnp.int32),
        pltpu.VMEM((NBUF, CHUNK, D), jnp.float32),
        pltpu.SemaphoreType.DMA((NBUF,)),
        pltpu.SemaphoreType.DMA((NBUF,)),
    ],
    compiler_params=pltpu.CompilerParams(use_tc_tiling_on_sc=False),
)
def _embed_gather(idx_hbm, table_hbm, out_hbm, idx_v, rows_v, gsem, ssem):
    wid = lax.axis_index("s") * NC + lax.axis_index("c")
    base = wid * B_PER_W

    def gather(g, b):
        return pltpu.make_async_copy(
            table_hbm.at[idx_v.at[g]], rows_v.at[b], gsem.at[b])

    def store(g, b):
        return pltpu.make_async_copy(
            rows_v.at[b], out_hbm.at[pl.ds(base + g * CHUNK, CHUNK)],
            ssem.at[b])

    pltpu.sync_copy(idx_hbm.at[pl.ds(wid * NSTEP, NSTEP)], idx_v)

    for b in range(NBUF):
        gather(b, b).start()

    @pl.loop(0, NSTEP - NBUF, step=NBUF)
    def _(g0):
        for b in range(NBUF):
            g = g0 + b
            gather(g, b).wait()
            store(g, b).start()
            store(g, b).wait()
            gather(g + NBUF, b).start()

    for b in range(NBUF):
        g = NSTEP - NBUF + b
        gather(g, b).wait()
        store(g, b).start()
    for b in range(NBUF):
        store(NSTEP - NBUF + b, b).wait()


def kernel(token_ids, weight):
    idx = token_ids.reshape(NW * NSTEP, CHUNK).astype(jnp.int32)
    out = _embed_gather(idx, weight)
    return out.reshape(*token_ids.shape, D)


# speedup vs baseline: 1.1149x; 1.0002x over previous
import functools

import jax
import jax.numpy as jnp
from jax import lax
from jax.experimental import pallas as pl
from jax.experimental.pallas import tpu as pltpu
from jax.experimental.pallas import tpu_sc as plsc

NC = 2
NS = 16
NW = NC * NS

B = 4096 * 200
D = 64
B_PER_W = B // NW
CHUNK = 128
NSTEP = B_PER_W // CHUNK
RING = 8
LAG = 4

_mesh = plsc.VectorSubcoreMesh(core_axis_name="c", subcore_axis_name="s")


@functools.partial(
    pl.kernel,
    out_type=jax.ShapeDtypeStruct((B, D), jnp.float32),
    mesh=_mesh,
    scratch_types=[
        pltpu.VMEM((NSTEP, CHUNK), jnp.int32),
        pltpu.VMEM((RING, CHUNK, D), jnp.float32),
        pltpu.SemaphoreType.DMA((RING,)),
        pltpu.SemaphoreType.DMA((RING,)),
    ],
    compiler_params=pltpu.CompilerParams(use_tc_tiling_on_sc=False),
)
def _embed_gather(idx_hbm, table_hbm, out_hbm, idx_v, rows_v, gsem, ssem):
    wid = lax.axis_index("s") * NC + lax.axis_index("c")
    base = wid * B_PER_W

    def gather(g, b):
        return pltpu.make_async_copy(
            table_hbm.at[idx_v.at[g]], rows_v.at[b], gsem.at[b])

    def store(g, b):
        return pltpu.make_async_copy(
            rows_v.at[b], out_hbm.at[pl.ds(base + g * CHUNK, CHUNK)],
            ssem.at[b])

    pltpu.sync_copy(idx_hbm.at[pl.ds(wid * NSTEP, NSTEP)], idx_v)

    for s in range(RING):
        gather(s, s).start()
        if s >= LAG:
            gather(s - LAG, s - LAG).wait()
            store(s - LAG, s - LAG).start()

    @pl.loop(RING, NSTEP, step=RING)
    def _(g0):
        for j in range(RING):
            g = g0 + j
            store(g - RING, j).wait()
            gather(g, j).start()
            bl = (j - LAG) % RING
            gather(g - LAG, bl).wait()
            store(g - LAG, bl).start()

    for t in range(LAG):
        g = NSTEP - LAG + t
        b = (NSTEP - LAG + t) % RING
        gather(g, b).wait()
        store(g, b).start()
    for j in range(RING):
        store(NSTEP - RING + j, j).wait()


def kernel(token_ids, weight):
    idx = token_ids.reshape(NW * NSTEP, CHUNK).astype(jnp.int32)
    out = _embed_gather(idx, weight)
    return out.reshape(*token_ids.shape, D)
